# baseline (speedup 1.0000x reference)
import functools

import jax
import jax.numpy as jnp
from jax import lax
from jax.experimental import pallas as pl
from jax.experimental.pallas import tpu as pltpu
from jax.experimental.pallas import tpu_sc as plsc

_NUM_CORES = 2
_NUM_SUBCORES = 16
_LANES = 16
_IDX_CHUNK = 128
_MV_BLK = 65536


@functools.lru_cache(maxsize=None)
def _build_matvec(D, N):
    grid = (N + _MV_BLK - 1) // _MV_BLK

    def body(w_ref, t_ref, z_ref):
        z_ref[...] = jnp.dot(w_ref[...], t_ref[...],
                             preferred_element_type=jnp.float32)

    return pl.pallas_call(
        body,
        grid=(grid,),
        in_specs=[
            pl.BlockSpec((D,), lambda i: (0,)),
            pl.BlockSpec((D, _MV_BLK), lambda i: (0, i)),
        ],
        out_specs=pl.BlockSpec((_MV_BLK,), lambda i: (i,)),
        out_shape=jax.ShapeDtypeStruct((N,), jnp.float32),
    )


@functools.lru_cache(maxsize=None)
def _build_sc_gather(B, b_per_w, n_chunks):
    mesh = plsc.VectorSubcoreMesh(
        core_axis_name="c",
        subcore_axis_name="s",
        num_cores=_NUM_CORES,
        num_subcores=_NUM_SUBCORES,
    )

    @functools.partial(
        pl.kernel,
        out_type=jax.ShapeDtypeStruct((B,), jnp.float32),
        mesh=mesh,
        compiler_params=pltpu.CompilerParams(
            needs_layout_passes=False, use_tc_tiling_on_sc=False),
        scratch_types=[
            pltpu.VMEM((b_per_w,), jnp.int32),
            pltpu.VMEM((b_per_w,), jnp.int32),
            pltpu.VMEM((b_per_w,), jnp.float32),
            pltpu.VMEM((b_per_w,), jnp.float32),
            pltpu.VMEM((_LANES,), jnp.float32),
            pltpu.VMEM((b_per_w,), jnp.float32),
            pltpu.SemaphoreType.DMA,
            pltpu.SemaphoreType.DMA,
        ],
    )
    def sc_kernel(uid_hbm, mid_hbm, zu_hbm, zm_hbm, wb_hbm, out_hbm,
                  uid_v, mid_v, zu_v, zm_v, wv, out_v, su, sm):
        wid = lax.axis_index("s") * _NUM_CORES + lax.axis_index("c")
        base = wid * b_per_w

        pltpu.sync_copy(uid_hbm.at[pl.ds(base, b_per_w)], uid_v)
        pltpu.sync_copy(mid_hbm.at[pl.ds(base, b_per_w)], mid_v)
        pltpu.sync_copy(wb_hbm, wv)

        copies = []
        for c in range(n_chunks):
            s = pl.ds(c * _IDX_CHUNK, _IDX_CHUNK)
            copies.append((
                pltpu.async_copy(zu_hbm.at[uid_v.at[s]], zu_v.at[s], su),
                pltpu.async_copy(zm_hbm.at[mid_v.at[s]], zm_v.at[s], sm),
            ))

        bias = wv[pl.ds(0, _LANES)][0]
        for cu, cm in copies:
            cu.wait()
            cm.wait()

        def group_body(g, _):
            s = pl.ds(g * _LANES, _LANES)
            acc = zu_v[s] + zm_v[s] + bias
            out_v[s] = 5.0 / (1.0 + jnp.exp(-acc))
            return 0

        lax.fori_loop(0, b_per_w // _LANES, group_body, 0)

        pltpu.sync_copy(out_v, out_hbm.at[pl.ds(base, b_per_w)])

    return sc_kernel


def kernel(user_ids, movie_ids, user_table, movie_table, fc_w, fc_b):
    B = user_ids.shape[0]
    D = user_table.shape[1]
    n_workers = _NUM_CORES * _NUM_SUBCORES
    b_per_w = B // n_workers
    n_chunks = b_per_w // _IDX_CHUNK

    w = fc_w.reshape(-1).astype(jnp.float32)
    w_u, w_m = w[:D], w[D:]
    wb = jnp.concatenate([fc_b.reshape(-1).astype(jnp.float32),
                          jnp.zeros((_LANES - 1,), jnp.float32)])

    zu = _build_matvec(D, user_table.shape[0])(w_u, user_table.T)
    zm = _build_matvec(D, movie_table.shape[0])(w_m, movie_table.T)

    sc = _build_sc_gather(B, b_per_w, n_chunks)
    out = sc(user_ids.astype(jnp.int32), movie_ids.astype(jnp.int32),
             zu, zm, wb)
    return out.reshape(B, 1)


# speedup vs baseline: 1.0210x; 1.0210x over previous
import functools

import jax
import jax.numpy as jnp
from jax import lax
from jax.experimental import pallas as pl
from jax.experimental.pallas import tpu as pltpu
from jax.experimental.pallas import tpu_sc as plsc

_NUM_CORES = 2
_NUM_SUBCORES = 16
_LANES = 16
_IDX_CHUNK = 128
_MV_BLK = 32768


@functools.lru_cache(maxsize=None)
def _build_matvec(D, N):
    grid = (N + _MV_BLK - 1) // _MV_BLK

    def body(w_ref, t_ref, z_ref):
        z_ref[...] = jnp.dot(w_ref[...], t_ref[...],
                             preferred_element_type=jnp.float32)

    return pl.pallas_call(
        body,
        grid=(grid,),
        in_specs=[
            pl.BlockSpec((D,), lambda i: (0,)),
            pl.BlockSpec((D, _MV_BLK), lambda i: (0, i)),
        ],
        out_specs=pl.BlockSpec((_MV_BLK,), lambda i: (i,)),
        out_shape=jax.ShapeDtypeStruct((N,), jnp.float32),
    )


@functools.lru_cache(maxsize=None)
def _build_sc_gather(B, b_per_w, n_chunks):
    mesh = plsc.VectorSubcoreMesh(
        core_axis_name="c",
        subcore_axis_name="s",
        num_cores=_NUM_CORES,
        num_subcores=_NUM_SUBCORES,
    )

    @functools.partial(
        pl.kernel,
        out_type=jax.ShapeDtypeStruct((B,), jnp.float32),
        mesh=mesh,
        compiler_params=pltpu.CompilerParams(
            needs_layout_passes=False, use_tc_tiling_on_sc=False),
        scratch_types=[
            pltpu.VMEM((b_per_w,), jnp.int32),
            pltpu.VMEM((b_per_w,), jnp.int32),
            pltpu.VMEM((b_per_w,), jnp.float32),
            pltpu.VMEM((b_per_w,), jnp.float32),
            pltpu.VMEM((_LANES,), jnp.float32),
            pltpu.VMEM((b_per_w,), jnp.float32),
            pltpu.SemaphoreType.DMA,
            pltpu.SemaphoreType.DMA,
        ],
    )
    def sc_kernel(uid_hbm, mid_hbm, zu_hbm, zm_hbm, wb_hbm, out_hbm,
                  uid_v, mid_v, zu_v, zm_v, wv, out_v, su, sm):
        wid = lax.axis_index("s") * _NUM_CORES + lax.axis_index("c")
        base = wid * b_per_w

        pltpu.sync_copy(uid_hbm.at[pl.ds(base, b_per_w)], uid_v)
        pltpu.sync_copy(mid_hbm.at[pl.ds(base, b_per_w)], mid_v)
        pltpu.sync_copy(wb_hbm, wv)

        copies = []
        for c in range(n_chunks):
            s = pl.ds(c * _IDX_CHUNK, _IDX_CHUNK)
            copies.append((
                pltpu.async_copy(zu_hbm.at[uid_v.at[s]], zu_v.at[s], su),
                pltpu.async_copy(zm_hbm.at[mid_v.at[s]], zm_v.at[s], sm),
            ))

        bias = wv[pl.ds(0, _LANES)][0]
        for cu, cm in copies:
            cu.wait()
            cm.wait()

        def group_body(g, _):
            s = pl.ds(g * _LANES, _LANES)
            acc = zu_v[s] + zm_v[s] + bias
            out_v[s] = 5.0 / (1.0 + jnp.exp(-acc))
            return 0

        lax.fori_loop(0, b_per_w // _LANES, group_body, 0)

        pltpu.sync_copy(out_v, out_hbm.at[pl.ds(base, b_per_w)])

    return sc_kernel


def kernel(user_ids, movie_ids, user_table, movie_table, fc_w, fc_b):
    B = user_ids.shape[0]
    D = user_table.shape[1]
    n_workers = _NUM_CORES * _NUM_SUBCORES
    b_per_w = B // n_workers
    n_chunks = b_per_w // _IDX_CHUNK

    w = fc_w.reshape(-1).astype(jnp.float32)
    w_u, w_m = w[:D], w[D:]
    wb = jnp.concatenate([fc_b.reshape(-1).astype(jnp.float32),
                          jnp.zeros((_LANES - 1,), jnp.float32)])

    zu = _build_matvec(D, user_table.shape[0])(w_u, user_table.T)
    zm = _build_matvec(D, movie_table.shape[0])(w_m, movie_table.T)

    sc = _build_sc_gather(B, b_per_w, n_chunks)
    out = sc(user_ids.astype(jnp.int32), movie_ids.astype(jnp.int32),
             zu, zm, wb)
    return out.reshape(B, 1)


# speedup vs baseline: 1.0534x; 1.0317x over previous
import functools

import jax
import jax.numpy as jnp
from jax import lax
from jax.experimental import pallas as pl
from jax.experimental.pallas import tpu as pltpu
from jax.experimental.pallas import tpu_sc as plsc

_NUM_CORES = 2
_NUM_SUBCORES = 16
_LANES = 16
_IDX_CHUNK = 128
_MV_BLK = 32768


@functools.lru_cache(maxsize=None)
def _build_matvec(D, N, w_half, add_bias):
    grid = (N + _MV_BLK - 1) // _MV_BLK

    def body(w_ref, b_ref, t_ref, z_ref):
        wh = w_ref[0, pl.ds(w_half * D, D)]
        z = jnp.dot(wh, t_ref[...], preferred_element_type=jnp.float32)
        if add_bias:
            z = z + b_ref[0]
        z_ref[...] = z

    return pl.pallas_call(
        body,
        grid=(grid,),
        in_specs=[
            pl.BlockSpec((1, 2 * D), lambda i: (0, 0)),
            pl.BlockSpec(memory_space=pltpu.SMEM),
            pl.BlockSpec((D, _MV_BLK), lambda i: (0, i)),
        ],
        out_specs=pl.BlockSpec((_MV_BLK,), lambda i: (i,)),
        out_shape=jax.ShapeDtypeStruct((N,), jnp.float32),
    )


@functools.lru_cache(maxsize=None)
def _build_sc_gather(B, b_per_w, n_chunks):
    mesh = plsc.VectorSubcoreMesh(
        core_axis_name="c",
        subcore_axis_name="s",
        num_cores=_NUM_CORES,
        num_subcores=_NUM_SUBCORES,
    )

    @functools.partial(
        pl.kernel,
        out_type=jax.ShapeDtypeStruct((B,), jnp.float32),
        mesh=mesh,
        compiler_params=pltpu.CompilerParams(
            needs_layout_passes=False, use_tc_tiling_on_sc=False),
        scratch_types=[
            pltpu.VMEM((b_per_w,), jnp.int32),
            pltpu.VMEM((b_per_w,), jnp.int32),
            pltpu.VMEM((b_per_w,), jnp.float32),
            pltpu.VMEM((b_per_w,), jnp.float32),
            pltpu.VMEM((b_per_w,), jnp.float32),
            pltpu.SemaphoreType.DMA,
            pltpu.SemaphoreType.DMA,
        ],
    )
    def sc_kernel(uid_hbm, mid_hbm, zu_hbm, zm_hbm, out_hbm,
                  uid_v, mid_v, zu_v, zm_v, out_v, su, sm):
        wid = lax.axis_index("s") * _NUM_CORES + lax.axis_index("c")
        base = wid * b_per_w

        pltpu.sync_copy(uid_hbm.at[pl.ds(base, b_per_w)], uid_v)
        pltpu.sync_copy(mid_hbm.at[pl.ds(base, b_per_w)], mid_v)

        copies = []
        for c in range(n_chunks):
            s = pl.ds(c * _IDX_CHUNK, _IDX_CHUNK)
            copies.append((
                pltpu.async_copy(zu_hbm.at[uid_v.at[s]], zu_v.at[s], su),
                pltpu.async_copy(zm_hbm.at[mid_v.at[s]], zm_v.at[s], sm),
            ))

        for cu, cm in copies:
            cu.wait()
            cm.wait()

        def group_body(g, _):
            s = pl.ds(g * _LANES, _LANES)
            acc = zu_v[s] + zm_v[s]
            out_v[s] = 5.0 / (1.0 + jnp.exp(-acc))
            return 0

        lax.fori_loop(0, b_per_w // _LANES, group_body, 0)

        pltpu.sync_copy(out_v, out_hbm.at[pl.ds(base, b_per_w)])

    return sc_kernel


def kernel(user_ids, movie_ids, user_table, movie_table, fc_w, fc_b):
    B = user_ids.shape[0]
    D = user_table.shape[1]
    n_workers = _NUM_CORES * _NUM_SUBCORES
    b_per_w = B // n_workers
    n_chunks = b_per_w // _IDX_CHUNK

    zu = _build_matvec(D, user_table.shape[0], 0, False)(
        fc_w, fc_b, user_table.T)
    zm = _build_matvec(D, movie_table.shape[0], 1, True)(
        fc_w, fc_b, movie_table.T)

    sc = _build_sc_gather(B, b_per_w, n_chunks)
    out = sc(user_ids.astype(jnp.int32), movie_ids.astype(jnp.int32),
             zu, zm)
    return out.reshape(B, 1)
